# baseline (speedup 1.0000x reference)
import functools

import jax
import jax.numpy as jnp
from jax import lax
from jax.experimental import pallas as pl
from jax.experimental.pallas import tpu as pltpu
from jax.experimental.pallas import tpu_sc as plsc

_B, _N, _DIM = 16, 1024, 384
_CB_SIZE, _CB_DIM = 1024, 32
_LN_EPS = 1e-5
_TOK = _B * _N
_BLK = 2048
_GRID = _TOK // _BLK
_LOSS_SCALE = 1.25 / (_TOK * _CB_DIM)


def _vq_tc_body(x_ref, g_ref, bn_ref, w1_ref, b1_ref, w2_ref, b2_ref, e_ref,
                idx_ref, pw_ref, loss_ref):
    i = pl.program_id(0)
    x = x_ref[...]
    mu = jnp.mean(x, axis=1, keepdims=True)
    var = jnp.mean((x - mu) ** 2, axis=1, keepdims=True)
    xn = (x - mu) / jnp.sqrt(var + _LN_EPS) * g_ref[...] + bn_ref[...]
    a = 0.5 * xn * (1.0 + lax.erf(xn * (2.0 ** -0.5)))
    z = lax.dot_general(a, w1_ref[...], (((1,), (0,)), ((), ())),
                        preferred_element_type=jnp.float32) + b1_ref[...]
    e = e_ref[...]
    e_sq = jnp.sum(e * e, axis=1, keepdims=True)
    e_aug = jnp.concatenate([e, -0.5 * e_sq], axis=1)
    z_sq = jnp.sum(z * z, axis=1, keepdims=True)
    z_aug = jnp.concatenate([z, jnp.ones_like(z_sq)], axis=1)
    u = lax.dot_general(e_aug, z_aug, (((1,), (1,)), ((), ())),
                        preferred_element_type=jnp.float32)
    u3 = u.reshape(_CB_SIZE // 8, 8, _BLK)
    acc_v = u3[0]
    acc_r = jnp.zeros((8, _BLK), jnp.float32)
    for r in range(1, _CB_SIZE // 8):
        v = u3[r]
        better = v > acc_v
        acc_v = jnp.maximum(acc_v, v)
        acc_r = jnp.where(better, jnp.float32(r), acc_r)
    acc_s = lax.broadcasted_iota(jnp.int32, (8, _BLK), 0).astype(jnp.float32)
    w = 8
    while w > 1:
        w //= 2
        av, bv = acc_v[:w], acc_v[w:]
        ar, br = acc_r[:w], acc_r[w:]
        as_, bs = acc_s[:w], acc_s[w:]
        take_b = (bv > av) | ((bv == av) & ((br < ar) | ((br == ar) & (bs < as_))))
        acc_v = jnp.where(take_b, bv, av)
        acc_r = jnp.where(take_b, br, ar)
        acc_s = jnp.where(take_b, bs, as_)
    idxf = acc_r * 8.0 + acc_s
    idx_ref[0, 0, :] = idxf.astype(jnp.int32).reshape(_BLK)
    dmin_sum = jnp.sum(z_sq) - 2.0 * jnp.sum(acc_v)

    @pl.when(i == 0)
    def _():
        loss_ref[...] = jnp.zeros((1, 1), jnp.float32)
        pw_ref[...] = lax.dot_general(e, w2_ref[...], (((1,), (0,)), ((), ())),
                                      preferred_element_type=jnp.float32) + b2_ref[...]

    loss_ref[...] += dmin_sum.reshape(1, 1)

    @pl.when(i == _GRID - 1)
    def _():
        loss_ref[...] = loss_ref[...] * _LOSS_SCALE


def _vq_tc(x2, ln_g, ln_b, W1, b1, W2, b2, embed):
    full = lambda shape: pl.BlockSpec(shape, lambda i: (0,) * len(shape))
    return pl.pallas_call(
        _vq_tc_body,
        grid=(_GRID,),
        in_specs=[
            pl.BlockSpec((_BLK, _DIM), lambda i: (i, 0)),
            full((1, _DIM)),
            full((1, _DIM)),
            full((_DIM, _CB_DIM)),
            full((1, _CB_DIM)),
            full((_CB_DIM, _DIM)),
            full((1, _DIM)),
            full((_CB_SIZE, _CB_DIM)),
        ],
        out_specs=[
            pl.BlockSpec((1, 1, _BLK), lambda i: (i, 0, 0)),
            full((_CB_SIZE, _DIM)),
            full((1, 1)),
        ],
        out_shape=[
            jax.ShapeDtypeStruct((_GRID, 1, _BLK), jnp.int32),
            jax.ShapeDtypeStruct((_CB_SIZE, _DIM), jnp.float32),
            jax.ShapeDtypeStruct((1, 1), jnp.float32),
        ],
    )(x2, ln_g, ln_b, W1, b1, W2, b2, embed)


_CHUNK = 128


def _sc_gather(table, idx):
    info = plsc.get_sparse_core_info()
    nw = info.num_cores * info.num_subcores
    bpw = _TOK // nw
    mesh = plsc.VectorSubcoreMesh(core_axis_name="c", subcore_axis_name="s")

    nch = bpw // _CHUNK

    @functools.partial(
        pl.kernel, mesh=mesh,
        out_type=jax.ShapeDtypeStruct((_TOK, _DIM), jnp.float32),
        scratch_types=[
            pltpu.VMEM((bpw,), jnp.int32),
            pltpu.VMEM((_CHUNK, _DIM), jnp.float32),
            pltpu.VMEM((_CHUNK, _DIM), jnp.float32),
            pltpu.SemaphoreType.DMA,
        ],
    )
    def k(table_hbm, idx_hbm, out_hbm, idx_v, rows_a, rows_b, gsem):
        wid = lax.axis_index("s") * info.num_cores + lax.axis_index("c")
        base = wid * bpw
        pltpu.sync_copy(idx_hbm.at[pl.ds(base, bpw)], idx_v)
        bufs = (rows_a, rows_b)
        pend = [None] * nch
        pend[0] = pltpu.async_copy(
            table_hbm.at[idx_v.at[pl.ds(0, _CHUNK)]], bufs[0], gsem)
        for c in range(nch):
            pend[c].wait()
            if c + 1 < nch:
                pend[c + 1] = pltpu.async_copy(
                    table_hbm.at[idx_v.at[pl.ds((c + 1) * _CHUNK, _CHUNK)]],
                    bufs[(c + 1) % 2], gsem)
            pltpu.sync_copy(bufs[c % 2], out_hbm.at[pl.ds(base + c * _CHUNK, _CHUNK)])

    return k(table, idx)


def kernel(x, ln_g, ln_b, W1, b1, W2, b2, embed):
    x2 = x.reshape(_TOK, _DIM)
    idx3, pw2, loss = _vq_tc(
        x2, ln_g.reshape(1, _DIM), ln_b.reshape(1, _DIM),
        W1, b1.reshape(1, _CB_DIM), W2, b2.reshape(1, _DIM), embed)
    idx_flat = idx3.reshape(_TOK)
    quantized = _sc_gather(pw2, idx_flat).reshape(_B, _N, _DIM)
    return quantized, idx3.reshape(_B, _N), loss[0, 0]


# speedup vs baseline: 1.0137x; 1.0137x over previous
import functools

import jax
import jax.numpy as jnp
from jax import lax
from jax.experimental import pallas as pl
from jax.experimental.pallas import tpu as pltpu
from jax.experimental.pallas import tpu_sc as plsc

_B, _N, _DIM = 16, 1024, 384
_CB_SIZE, _CB_DIM = 1024, 32
_LN_EPS = 1e-5
_TOK = _B * _N
_BLK = 4096
_GRID = _TOK // _BLK
_LOSS_SCALE = 1.25 / (_TOK * _CB_DIM)


def _vq_tc_body(x_ref, g_ref, bn_ref, w1_ref, b1_ref, w2_ref, b2_ref, e_ref,
                idx_ref, pw_ref, loss_ref):
    i = pl.program_id(0)
    x = x_ref[...]
    mu = jnp.mean(x, axis=1, keepdims=True)
    var = jnp.mean((x - mu) ** 2, axis=1, keepdims=True)
    xn = (x - mu) / jnp.sqrt(var + _LN_EPS) * g_ref[...] + bn_ref[...]
    a = 0.5 * xn * (1.0 + lax.erf(xn * (2.0 ** -0.5)))
    z = lax.dot_general(a, w1_ref[...], (((1,), (0,)), ((), ())),
                        preferred_element_type=jnp.float32) + b1_ref[...]
    e = e_ref[...]
    e_sq = jnp.sum(e * e, axis=1, keepdims=True)
    e_aug = jnp.concatenate([e, -0.5 * e_sq], axis=1)
    z_sq = jnp.sum(z * z, axis=1, keepdims=True)
    z_aug = jnp.concatenate([z, jnp.ones_like(z_sq)], axis=1)
    u = lax.dot_general(e_aug, z_aug, (((1,), (1,)), ((), ())),
                        preferred_element_type=jnp.float32)
    u3 = u.reshape(_CB_SIZE // 8, 8, _BLK)
    acc_v = u3[0]
    acc_r = jnp.zeros((8, _BLK), jnp.float32)
    for r in range(1, _CB_SIZE // 8):
        v = u3[r]
        better = v > acc_v
        acc_v = jnp.maximum(acc_v, v)
        acc_r = jnp.where(better, jnp.float32(r), acc_r)
    acc_s = lax.broadcasted_iota(jnp.int32, (8, _BLK), 0).astype(jnp.float32)
    w = 8
    while w > 1:
        w //= 2
        av, bv = acc_v[:w], acc_v[w:]
        ar, br = acc_r[:w], acc_r[w:]
        as_, bs = acc_s[:w], acc_s[w:]
        take_b = (bv > av) | ((bv == av) & ((br < ar) | ((br == ar) & (bs < as_))))
        acc_v = jnp.where(take_b, bv, av)
        acc_r = jnp.where(take_b, br, ar)
        acc_s = jnp.where(take_b, bs, as_)
    idxf = acc_r * 8.0 + acc_s
    idx_ref[0, 0, :] = idxf.astype(jnp.int32).reshape(_BLK)
    dmin_sum = jnp.sum(z_sq) - 2.0 * jnp.sum(acc_v)

    @pl.when(i == 0)
    def _():
        loss_ref[...] = jnp.zeros((1, 1), jnp.float32)
        pw_ref[...] = lax.dot_general(e, w2_ref[...], (((1,), (0,)), ((), ())),
                                      preferred_element_type=jnp.float32) + b2_ref[...]

    loss_ref[...] += dmin_sum.reshape(1, 1)

    @pl.when(i == _GRID - 1)
    def _():
        loss_ref[...] = loss_ref[...] * _LOSS_SCALE


def _vq_tc(x2, ln_g, ln_b, W1, b1, W2, b2, embed):
    full = lambda shape: pl.BlockSpec(shape, lambda i: (0,) * len(shape))
    return pl.pallas_call(
        _vq_tc_body,
        grid=(_GRID,),
        in_specs=[
            pl.BlockSpec((_BLK, _DIM), lambda i: (i, 0)),
            full((1, _DIM)),
            full((1, _DIM)),
            full((_DIM, _CB_DIM)),
            full((1, _CB_DIM)),
            full((_CB_DIM, _DIM)),
            full((1, _DIM)),
            full((_CB_SIZE, _CB_DIM)),
        ],
        out_specs=[
            pl.BlockSpec((1, 1, _BLK), lambda i: (i, 0, 0)),
            full((_CB_SIZE, _DIM)),
            full((1, 1)),
        ],
        out_shape=[
            jax.ShapeDtypeStruct((_GRID, 1, _BLK), jnp.int32),
            jax.ShapeDtypeStruct((_CB_SIZE, _DIM), jnp.float32),
            jax.ShapeDtypeStruct((1, 1), jnp.float32),
        ],
    )(x2, ln_g, ln_b, W1, b1, W2, b2, embed)


_CHUNK = 128


def _sc_gather(table, idx):
    info = plsc.get_sparse_core_info()
    nw = info.num_cores * info.num_subcores
    bpw = _TOK // nw
    mesh = plsc.VectorSubcoreMesh(core_axis_name="c", subcore_axis_name="s")

    nch = bpw // _CHUNK

    @functools.partial(
        pl.kernel, mesh=mesh,
        out_type=jax.ShapeDtypeStruct((_TOK, _DIM), jnp.float32),
        scratch_types=[
            pltpu.VMEM((bpw,), jnp.int32),
            pltpu.VMEM((_CHUNK, _DIM), jnp.float32),
            pltpu.VMEM((_CHUNK, _DIM), jnp.float32),
            pltpu.SemaphoreType.DMA,
        ],
    )
    def k(table_hbm, idx_hbm, out_hbm, idx_v, rows_a, rows_b, gsem):
        wid = lax.axis_index("s") * info.num_cores + lax.axis_index("c")
        base = wid * bpw
        pltpu.sync_copy(idx_hbm.at[pl.ds(base, bpw)], idx_v)
        bufs = (rows_a, rows_b)
        pend = [None] * nch
        pend[0] = pltpu.async_copy(
            table_hbm.at[idx_v.at[pl.ds(0, _CHUNK)]], bufs[0], gsem)
        for c in range(nch):
            pend[c].wait()
            if c + 1 < nch:
                pend[c + 1] = pltpu.async_copy(
                    table_hbm.at[idx_v.at[pl.ds((c + 1) * _CHUNK, _CHUNK)]],
                    bufs[(c + 1) % 2], gsem)
            pltpu.sync_copy(bufs[c % 2], out_hbm.at[pl.ds(base + c * _CHUNK, _CHUNK)])

    return k(table, idx)


def kernel(x, ln_g, ln_b, W1, b1, W2, b2, embed):
    x2 = x.reshape(_TOK, _DIM)
    idx3, pw2, loss = _vq_tc(
        x2, ln_g.reshape(1, _DIM), ln_b.reshape(1, _DIM),
        W1, b1.reshape(1, _CB_DIM), W2, b2.reshape(1, _DIM), embed)
    idx_flat = idx3.reshape(_TOK)
    quantized = _sc_gather(pw2, idx_flat).reshape(_B, _N, _DIM)
    return quantized, idx3.reshape(_B, _N), loss[0, 0]
